# Initial kernel scaffold; baseline (speedup 1.0000x reference)
import jax
import jax.numpy as jnp
from jax.experimental import pallas as pl


def kernel(adj_index, adj_vals, d1_index, d1_vals, uEmbeds, iEmbeds):
    raise NotImplementedError("write your pallas kernel here")


# speedup vs baseline: 3.9816x; 3.9816x over previous
import jax
import jax.numpy as jnp
from jax import lax
from jax.experimental import pallas as pl
from jax.experimental.pallas import tpu as pltpu
from jax.experimental.pallas import tpu_sc as plsc

_USER = 5000
_N = 10000
_NP = 10240
_DH = 64
_E = 320000
_LANES = 128
_RPW = 4
_WIN = 40
_RPS = _WIN * _RPW
_EPAD = 16 * _RPS * _LANES
_EROWS = _EPAD // _LANES
_NROW = _NP // 16


def _pack_edges(index, vals):
    pad = _EPAD - _E
    ar = jnp.arange(pad, dtype=jnp.int32)
    rows = jnp.concatenate([index[0], ar % _N]).reshape(_EROWS, _LANES)
    cols = jnp.concatenate([index[1], (ar * 7919) % _N]).reshape(_EROWS, _LANES)
    v = jnp.concatenate([vals, jnp.zeros((pad,), vals.dtype)])
    return jnp.stack([rows, cols], axis=1), v


def _spmm_pass(sid, coff, edg_hbm, val_hbm, src_hbm, dst_sh,
               edg_v, val_v, gath, sem):
    def win(w, carry):
        base = sid * _RPS + w * _RPW
        pltpu.sync_copy(edg_hbm.at[pl.ds(base, _RPW)], edg_v)
        pltpu.sync_copy(val_hbm.at[pl.ds(base * _LANES, _RPW * _LANES)], val_v)
        for j in range(_RPW):
            for i in range(_LANES // 16):
                sl = (j, 1, pl.ds(i * 16, 16))
                edg_v[sl] = edg_v[sl] + coff
        descs = [
            pltpu.async_copy(src_hbm.at[edg_v.at[j, 1]], gath.at[j], sem)
            for j in range(_RPW)
        ]
        for d in descs:
            d.wait()
        for j in range(_RPW):
            def escale(g, c2, j=j):
                vv = val_v[pl.ds(j * _LANES + g * 16, 16)]
                for e2 in range(16):
                    vf = jnp.broadcast_to(vv[e2], (16,))
                    e = g * 16 + e2
                    for k in range(_DH // 16):
                        sl = (e, pl.ds(k * 16, 16))
                        gath[(j,) + sl] = gath[(j,) + sl] * vf
                return c2
            lax.fori_loop(0, _LANES // 16, escale, 0)
        for j in range(_RPW):
            pltpu.sync_copy(gath.at[j], dst_sh.at[edg_v.at[j, 0]], add=True)
        return carry

    lax.fori_loop(0, _WIN, win, 0)


def _body(edges_a, vals_a, edges_d, vals_d, xcat, zeros,
          mean_out, cond_out, h1_out,
          acc, edg_v, val_v, gath, bx, bh, b2, sem):
    cid = lax.axis_index("c")
    sid = lax.axis_index("s")
    coff = cid * _NP
    rbase = sid * _NROW

    pltpu.sync_copy(zeros, acc.at[pl.ds(rbase, _NROW)])
    plsc.subcore_barrier()

    _spmm_pass(sid, coff, edges_a, vals_a, xcat, acc, edg_v, val_v, gath, sem)
    plsc.subcore_barrier()
    pltpu.sync_copy(acc.at[pl.ds(rbase, _NROW)],
                    h1_out.at[pl.ds(coff + rbase, _NROW)])
    plsc.subcore_barrier()
    pltpu.sync_copy(zeros, acc.at[pl.ds(rbase, _NROW)])
    plsc.subcore_barrier()

    _spmm_pass(sid, coff, edges_a, vals_a, h1_out, acc, edg_v, val_v, gath,
               sem)
    plsc.subcore_barrier()

    for t in range(5):
        r0 = rbase + t * 128
        pltpu.sync_copy(xcat.at[pl.ds(coff + r0, 128)], bx)
        pltpu.sync_copy(h1_out.at[pl.ds(coff + r0, 128)], bh)
        pltpu.sync_copy(acc.at[pl.ds(r0, 128)], b2)

        def mrow(r, c2):
            for k in range(_DH // 16):
                sl = (r, pl.ds(k * 16, 16))
                bx[sl] = (bx[sl] + bh[sl] + b2[sl]) * jnp.float32(1.0 / 3.0)
            return c2

        lax.fori_loop(0, 128, mrow, 0, unroll=4)
        pltpu.sync_copy(bx, mean_out.at[pl.ds(coff + r0, 128)])
    plsc.subcore_barrier()
    pltpu.sync_copy(zeros, acc.at[pl.ds(rbase, _NROW)])
    plsc.subcore_barrier()

    _spmm_pass(sid, coff, edges_d, vals_d, mean_out, acc, edg_v, val_v, gath,
               sem)
    plsc.subcore_barrier()
    pltpu.sync_copy(acc.at[pl.ds(rbase, _NROW)],
                    cond_out.at[pl.ds(coff + rbase, _NROW)])


def kernel(adj_index, adj_vals, d1_index, d1_vals, uEmbeds, iEmbeds):
    embeds = jnp.concatenate([uEmbeds, iEmbeds], axis=0)
    zpad = jnp.zeros((_NP - _N, _DH), jnp.float32)
    xcat = jnp.concatenate(
        [embeds[:, :_DH], zpad, embeds[:, _DH:], zpad], axis=0)
    edges_a, vals_a = _pack_edges(adj_index, adj_vals)
    edges_d, vals_d = _pack_edges(d1_index, d1_vals)
    zeros = jnp.zeros((_NROW, _DH), jnp.float32)

    mesh = plsc.VectorSubcoreMesh(core_axis_name="c", subcore_axis_name="s",
                                  num_cores=2, num_subcores=16)
    f32 = jnp.float32
    half = jax.ShapeDtypeStruct((2 * _NP, _DH), f32)
    call = pl.kernel(
        _body,
        out_type=(half, half, half),
        mesh=mesh,
        compiler_params=pltpu.CompilerParams(use_tc_tiling_on_sc=False),
        scratch_types=[
            pltpu.VMEM_SHARED((_NP, _DH), f32),
            pltpu.VMEM((_RPW, 2, _LANES), jnp.int32),
            pltpu.VMEM((_RPW * _LANES,), f32),
            pltpu.VMEM((_RPW, _LANES, _DH), f32),
            pltpu.VMEM((128, _DH), f32),
            pltpu.VMEM((128, _DH), f32),
            pltpu.VMEM((128, _DH), f32),
            pltpu.SemaphoreType.DMA,
        ],
    )
    mean_h, cond_h, _ = call(edges_a, vals_a, edges_d, vals_d, xcat, zeros)
    mean = jnp.concatenate([mean_h[:_N], mean_h[_NP:_NP + _N]], axis=1)
    cond = jnp.concatenate([cond_h[:_N], cond_h[_NP:_NP + _N]], axis=1)
    return mean[:_USER], mean[_USER:], cond, uEmbeds, iEmbeds
